# baseline (speedup 1.0000x reference)
import jax
import jax.numpy as jnp
import numpy as np
from jax.experimental import pallas as pl

_N_FEATURES = 128
_P_SINGLE = 0.3
_P_DOUBLE = 0.15


def _rng_inputs(batch_size: int):
    with jax.ensure_compile_time_eval():
        key = jax.random.key(42)
        k1, k2 = jax.random.split(key)
        r = jax.random.uniform(k1, (batch_size,))
        n_to_drop = jnp.where(
            r < _P_DOUBLE, 2, jnp.where(r < _P_SINGLE + _P_DOUBLE, 1, 0)
        ).astype(jnp.int32)
        u = jax.random.uniform(k2, (batch_size, _N_FEATURES))
        n_b = jnp.broadcast_to(n_to_drop[:, None], (batch_size, _N_FEATURES))
        return np.asarray(u), np.asarray(n_b)


def _apply_kernel(u_ref, n_ref, x_ref, o_ref):
    u = u_ref[...]
    b, f = u.shape
    n = n_ref[...]
    ii = jax.lax.broadcasted_iota(jnp.int32, (b, f), 1)
    big = jnp.int32(f)
    m1 = jnp.min(u, axis=1, keepdims=True)
    i1 = jnp.min(jnp.where(u == m1, ii, big), axis=1, keepdims=True)
    is1 = ii == i1
    u2 = jnp.where(is1, jnp.inf, u)
    m2 = jnp.min(u2, axis=1, keepdims=True)
    i2 = jnp.min(jnp.where(u2 == m2, ii, big), axis=1, keepdims=True)
    is2 = ii == i2
    drop = (is1 & (n >= 1)) | (is2 & (n >= 2))
    mask = jnp.where(drop, 0.0, 1.0)
    o_ref[...] = x_ref[...] * mask[:, None, :]


def kernel(x):
    batch, seq, feat = x.shape
    u, n_b = _rng_inputs(batch)
    blk = 32
    grid = (batch // blk,)
    return pl.pallas_call(
        _apply_kernel,
        grid=grid,
        in_specs=[
            pl.BlockSpec((blk, feat), lambda i: (i, 0)),
            pl.BlockSpec((blk, feat), lambda i: (i, 0)),
            pl.BlockSpec((blk, seq, feat), lambda i: (i, 0, 0)),
        ],
        out_specs=pl.BlockSpec((blk, seq, feat), lambda i: (i, 0, 0)),
        out_shape=jax.ShapeDtypeStruct(x.shape, x.dtype),
    )(u, n_b, x)


# speedup vs baseline: 1.0803x; 1.0803x over previous
import jax
import jax.numpy as jnp
import numpy as np
from jax.experimental import pallas as pl

_N_FEATURES = 128
_P_SINGLE = 0.3
_P_DOUBLE = 0.15


def _rng_inputs(batch_size: int):
    with jax.ensure_compile_time_eval():
        key = jax.random.key(42)
        k1, k2 = jax.random.split(key)
        r = jax.random.uniform(k1, (batch_size,))
        n_to_drop = jnp.where(
            r < _P_DOUBLE, 2, jnp.where(r < _P_SINGLE + _P_DOUBLE, 1, 0)
        ).astype(jnp.int32)
        u = jax.random.uniform(k2, (batch_size, _N_FEATURES))
        n_b = jnp.broadcast_to(n_to_drop[:, None], (batch_size, _N_FEATURES))
        return np.asarray(u), np.asarray(n_b)


def _apply_kernel(u_ref, n_ref, x_ref, o_ref):
    u = u_ref[...]
    b, f = u.shape
    n = n_ref[...]
    ii = jax.lax.broadcasted_iota(jnp.int32, (b, f), 1)
    big = jnp.int32(f)
    m1 = jnp.min(u, axis=1, keepdims=True)
    i1 = jnp.min(jnp.where(u == m1, ii, big), axis=1, keepdims=True)
    is1 = ii == i1
    u2 = jnp.where(is1, jnp.inf, u)
    m2 = jnp.min(u2, axis=1, keepdims=True)
    i2 = jnp.min(jnp.where(u2 == m2, ii, big), axis=1, keepdims=True)
    is2 = ii == i2
    drop = (is1 & (n >= 1)) | (is2 & (n >= 2))
    mask = jnp.where(drop, 0.0, 1.0)
    o_ref[...] = x_ref[...] * mask[:, None, :]


def kernel(x):
    batch, seq, feat = x.shape
    u, n_b = _rng_inputs(batch)
    blk = 128
    grid = (batch // blk,)
    return pl.pallas_call(
        _apply_kernel,
        grid=grid,
        in_specs=[
            pl.BlockSpec((blk, feat), lambda i: (i, 0)),
            pl.BlockSpec((blk, feat), lambda i: (i, 0)),
            pl.BlockSpec((blk, seq, feat), lambda i: (i, 0, 0)),
        ],
        out_specs=pl.BlockSpec((blk, seq, feat), lambda i: (i, 0, 0)),
        out_shape=jax.ShapeDtypeStruct(x.shape, x.dtype),
    )(u, n_b, x)
